# Initial kernel scaffold; baseline (speedup 1.0000x reference)
import jax
import jax.numpy as jnp
from jax.experimental import pallas as pl


def kernel(imp):
    raise NotImplementedError("write your pallas kernel here")


# speedup vs baseline: 67.1532x; 67.1532x over previous
import functools

import jax
import jax.numpy as jnp
from jax.experimental import pallas as pl
from jax.experimental.pallas import tpu as pltpu

_RATE = 0.75


def _select_body(k, x_ref, mask_ref, cnt_ref):
    x = x_ref[0]
    i = jax.lax.bitcast_convert_type(x, jnp.int32)
    key = i ^ ((i >> 31) & jnp.int32(0x7FFFFFFF))

    c0 = jnp.sum((key >= 0).astype(jnp.int32))
    p0 = jnp.where(c0 >= k, jnp.int32(0), jnp.int32(-(2**31)))

    def step(b, p):
        t = p + (jnp.int32(1) << (jnp.int32(30) - b))
        c = jnp.sum((key >= t).astype(jnp.int32))
        return jnp.where(c >= k, t, p)

    p = jax.lax.fori_loop(0, 31, step, p0)

    gt = key > p
    cg = jnp.sum(gt.astype(jnp.int32))
    need = k - cg
    tie = key == p
    tie_f = tie.astype(jnp.float32)
    H, W = tie_f.shape
    uw = (
        jax.lax.broadcasted_iota(jnp.int32, (W, W), 0)
        <= jax.lax.broadcasted_iota(jnp.int32, (W, W), 1)
    ).astype(jnp.float32)
    row_c = jnp.dot(tie_f, uw, preferred_element_type=jnp.float32)
    row_tot = row_c[:, W - 1 :]
    lh = (
        jax.lax.broadcasted_iota(jnp.int32, (H, H), 1)
        < jax.lax.broadcasted_iota(jnp.int32, (H, H), 0)
    ).astype(jnp.float32)
    offs = jnp.dot(lh, row_tot, preferred_element_type=jnp.float32)
    rank = row_c + offs
    sel = tie & (rank <= jnp.float32(1.0) * need.astype(jnp.float32))
    m = jnp.where(gt | sel, jnp.float32(1.0), jnp.float32(0.0))
    mask_ref[0] = m
    cnt_ref[0, 0, 0] = jnp.sum(m)


@jax.jit
def kernel(imp):
    B, H, W = imp.shape
    k = max(1, int(_RATE * H * W))
    mask, cnt = pl.pallas_call(
        functools.partial(_select_body, k),
        grid=(B,),
        in_specs=[pl.BlockSpec((1, H, W), lambda b: (b, 0, 0))],
        out_specs=[
            pl.BlockSpec((1, H, W), lambda b: (b, 0, 0)),
            pl.BlockSpec((1, 1, 1), lambda b: (b, 0, 0), memory_space=pltpu.SMEM),
        ],
        out_shape=[
            jax.ShapeDtypeStruct((B, H, W), jnp.float32),
            jax.ShapeDtypeStruct((B, 1, 1), jnp.float32),
        ],
    )(imp)
    mean = jnp.sum(cnt) / jnp.float32(B * H * W)
    return mask[:, None, :, :], mean
